# baseline (speedup 1.0000x reference)
import functools

import jax
import jax.numpy as jnp
from jax import lax
from jax.experimental import pallas as pl
from jax.experimental.pallas import tpu as pltpu
from jax.experimental.pallas import tpu_sc as plsc

_N = 10000
_E = 160000
_C = 64

_NC = 2
_NS = 16
_NW = _NC * _NS
_EPW = _E // _NW
_CH = 128
_NFULL = _EPW // _CH
_TAIL = _EPW - _NFULL * _CH
_RPT = _N // _NS

_HALVES = ((0, _E),)


def _make_gather_body(base, epw):
    nfull, tail = epw // _CH, epw % _CH
    assert nfull % 2 == 1 and tail > 0

    def body(x_hbm, idx_hbm, out_hbm, idx_all, rows0, rows1, rows_t,
             sem0, sem1):
        c = lax.axis_index("c")
        s = lax.axis_index("s")
        wid = s * _NC + c
        wbase = base + wid * epw
        pltpu.sync_copy(idx_hbm.at[pl.ds(pl.multiple_of(wbase, 8), epw)],
                        idx_all)

        def gst(j, buf, sem):
            pltpu.async_copy(x_hbm.at[idx_all.at[pl.ds(j * _CH, _CH)]],
                             buf, sem)

        def gwt(buf, sem):
            pltpu.make_async_copy(
                x_hbm.at[idx_all.at[pl.ds(0, _CH)]], buf, sem).wait()

        def wb(j, buf):
            pltpu.sync_copy(
                buf, out_hbm.at[pl.ds(pl.multiple_of(wbase + j * _CH, 8),
                                      _CH)])

        gst(0, rows0, sem0)

        def pair(p, carry):
            j = 2 * p
            gst(j + 1, rows1, sem1)
            gwt(rows0, sem0)
            wb(j, rows0)
            gst(j + 2, rows0, sem0)
            gwt(rows1, sem1)
            wb(j + 1, rows1)
            return carry

        lax.fori_loop(0, nfull // 2, pair, 0)
        gwt(rows0, sem0)
        wb(nfull - 1, rows0)
        offt = pl.multiple_of(wbase + nfull * _CH, 8)
        pltpu.async_copy(x_hbm.at[idx_all.at[pl.ds(nfull * _CH, tail)]],
                         rows_t, sem0).wait()
        pltpu.sync_copy(rows_t, out_hbm.at[pl.ds(offt, tail)])

    return body, tail


def _make_scatter_body(base, epw):
    nfull, tail = epw // _CH, epw % _CH
    assert nfull % 2 == 1 and tail > 0

    def body(msg_hbm, dst_hbm, zero_hbm, parts_hbm, idx0, idx1, rows0, rows1,
             idx_t, rows_t, agg_sh, si0, si1, sm0, sm1):
        c = lax.axis_index("c")
        s = lax.axis_index("s")
        wid = s * _NC + c
        wbase = base + wid * epw

        def lst(j, ib, rb, si, sm):
            off = pl.multiple_of(wbase + j * _CH, 8)
            pltpu.async_copy(dst_hbm.at[pl.ds(off, _CH)], ib, si)
            pltpu.async_copy(msg_hbm.at[pl.ds(off, _CH)], rb, sm)

        def lwt(ib, rb, si, sm):
            off0 = pl.multiple_of(wbase, 8)
            pltpu.make_async_copy(dst_hbm.at[pl.ds(off0, _CH)], ib, si).wait()
            pltpu.make_async_copy(msg_hbm.at[pl.ds(off0, _CH)], rb, sm).wait()

        lst(0, idx0, rows0, si0, sm0)
        pltpu.sync_copy(zero_hbm, agg_sh.at[pl.ds(s * _RPT, _RPT)])
        plsc.subcore_barrier()

        def pair(p, carry):
            j = 2 * p
            lst(j + 1, idx1, rows1, si1, sm1)
            lwt(idx0, rows0, si0, sm0)
            pltpu.sync_copy(rows0, agg_sh.at[idx0], add=True)
            lst(j + 2, idx0, rows0, si0, sm0)
            lwt(idx1, rows1, si1, sm1)
            pltpu.sync_copy(rows1, agg_sh.at[idx1], add=True)
            return carry

        lax.fori_loop(0, nfull // 2, pair, 0)
        lwt(idx0, rows0, si0, sm0)
        pltpu.sync_copy(rows0, agg_sh.at[idx0], add=True)
        offt = pl.multiple_of(wbase + nfull * _CH, 8)
        pltpu.sync_copy(dst_hbm.at[pl.ds(offt, tail)], idx_t)
        pltpu.sync_copy(msg_hbm.at[pl.ds(offt, tail)], rows_t)
        pltpu.sync_copy(rows_t, agg_sh.at[idx_t], add=True)
        plsc.subcore_barrier()
        pltpu.sync_copy(agg_sh.at[pl.ds(s * _RPT, _RPT)],
                        parts_hbm.at[pl.ds(c * _N + s * _RPT, _RPT)])

    return body, tail


@functools.lru_cache(maxsize=None)
def _sc_kernels():
    mesh = plsc.VectorSubcoreMesh(core_axis_name="c", subcore_axis_name="s")
    params = pltpu.CompilerParams(use_tc_tiling_on_sc=False)
    kernels = []
    for base, n_edges in _HALVES:
        epw = n_edges // _NW
        gbody, gtail = _make_gather_body(base, epw)
        gather = pl.kernel(
            gbody,
            mesh=mesh,
            compiler_params=params,
            out_type=jax.ShapeDtypeStruct((_E, _C), jnp.bfloat16),
            scratch_types=[
                pltpu.VMEM((epw,), jnp.int32),
                pltpu.VMEM((_CH, _C), jnp.bfloat16),
                pltpu.VMEM((_CH, _C), jnp.bfloat16),
                pltpu.VMEM((max(gtail, 8), _C), jnp.bfloat16),
                pltpu.SemaphoreType.DMA,
                pltpu.SemaphoreType.DMA,
            ],
        )
        sbody, stail = _make_scatter_body(base, epw)
        scatter = pl.kernel(
            sbody,
            mesh=mesh,
            compiler_params=params,
            out_type=jax.ShapeDtypeStruct((2 * _N, _C), jnp.float32),
            scratch_types=[
                pltpu.VMEM((_CH,), jnp.int32),
                pltpu.VMEM((_CH,), jnp.int32),
                pltpu.VMEM((_CH, _C), jnp.float32),
                pltpu.VMEM((_CH, _C), jnp.float32),
                pltpu.VMEM((max(stail, 8),), jnp.int32),
                pltpu.VMEM((max(stail, 8), _C), jnp.float32),
                pltpu.VMEM_SHARED((_N, _C), jnp.float32),
                pltpu.SemaphoreType.DMA,
                pltpu.SemaphoreType.DMA,
                pltpu.SemaphoreType.DMA,
                pltpu.SemaphoreType.DMA,
            ],
        )
        kernels.append((gather, scatter))
    return kernels


_BE = 2048
_BH = 256


def _edge_half(ea, xj, W2T):
    eaT = ea.T
    xjT = xj.T
    rep = jnp.broadcast_to(xjT.astype(jnp.float32)[:, None, :],
                           (_C, _C, _BH)).reshape(_C * _C, _BH)
    eaT_t = jnp.tile(eaT, (_C, 1))
    u = (rep * eaT_t).astype(jnp.bfloat16)
    u_full = jnp.concatenate([u, xjT], axis=0)
    msgT = jnp.dot(W2T, u_full,
                   preferred_element_type=jnp.float32)
    return msgT.T


def _edge_body(ea_ref, xj_ref, W2T_ref, msg_ref):
    W2T = W2T_ref[...]
    for h in range(_BE // _BH):
        sl = pl.ds(h * _BH, _BH)
        msg_ref[sl, :] = _edge_half(ea_ref[sl, :], xj_ref[sl, :], W2T)


def _edge_call(ea, xj, W2T_bf16, blk0, n_edges):
    bmap = lambda i, b=blk0: (i + b, 0)
    return pl.pallas_call(
        _edge_body,
        grid=(pl.cdiv(n_edges, _BE),),
        in_specs=[
            pl.BlockSpec((_BE, _C), bmap),
            pl.BlockSpec((_BE, _C), bmap),
            pl.BlockSpec((_C, _C * _C + _C), lambda i: (0, 0)),
        ],
        out_specs=pl.BlockSpec((_BE, _C), bmap),
        out_shape=jax.ShapeDtypeStruct((_E, _C), jnp.float32),
    )(ea, xj, W2T_bf16)


_BN = 2000


def _combine_body(p_ref, x_ref, lwT_ref, b_ref, o_ref):
    s_ = (p_ref[0] + p_ref[1] + b_ref[...]
          + jnp.dot(x_ref[...], lwT_ref[...], preferred_element_type=jnp.float32))
    o_ref[...] = jnp.maximum(s_, 0.0)


def _combine_call(parts, x, lwT, b_row):
    return pl.pallas_call(
        _combine_body,
        grid=(_N // _BN,),
        in_specs=[
            pl.BlockSpec((2, _BN, _C), lambda i: (0, i, 0)),
            pl.BlockSpec((_BN, _C), lambda i: (i, 0)),
            pl.BlockSpec((_C, _C), lambda i: (0, 0)),
            pl.BlockSpec((1, _C), lambda i: (0, 0)),
        ],
        out_specs=pl.BlockSpec((_BN, _C), lambda i: (i, 0)),
        out_shape=jax.ShapeDtypeStruct((_N, _C), jnp.float32),
    )(parts, x, lwT, b_row)


def _combine_final_body(p_ref, x_ref, lwT_ref, b_ref, woT_ref, bo_ref, o_ref):
    s_ = (p_ref[0] + p_ref[1] + b_ref[...]
          + jnp.dot(x_ref[...], lwT_ref[...], preferred_element_type=jnp.float32))
    h = jnp.maximum(s_, 0.0)
    o_ref[...] = jnp.dot(h, woT_ref[...],
                         preferred_element_type=jnp.float32) + bo_ref[...]


def _combine_final_call(parts, x, lwT, b_row, woT, bo_row):
    return pl.pallas_call(
        _combine_final_body,
        grid=(_N // _BN,),
        in_specs=[
            pl.BlockSpec((2, _BN, _C), lambda i: (0, i, 0)),
            pl.BlockSpec((_BN, _C), lambda i: (i, 0)),
            pl.BlockSpec((_C, _C), lambda i: (0, 0)),
            pl.BlockSpec((1, _C), lambda i: (0, 0)),
            pl.BlockSpec((_C, _C), lambda i: (0, 0)),
            pl.BlockSpec((1, _C), lambda i: (0, 0)),
        ],
        out_specs=pl.BlockSpec((_BN, _C), lambda i: (i, 0)),
        out_shape=jax.ShapeDtypeStruct((_N, _C), jnp.float32),
    )(parts, x, lwT, b_row, woT, bo_row)


def kernel(feature, edge_index, edge_attr, nn_W0, nn_b0, lin_W0, bias0,
           nn_W1, nn_b1, lin_W1, bias1, W_out, b_out):
    src = edge_index[0]
    dst = edge_index[1]
    zero_init = jnp.zeros((_RPT, _C), jnp.float32)
    halves = _sc_kernels()

    def layer(x, nn_W, nn_b):
        W2T = nn_W.reshape(_C, _C, _C).transpose(1, 0, 2).reshape(_C, _C * _C)
        nbmT = nn_b.reshape(_C, _C).T
        W2T = jnp.concatenate([W2T, nbmT], axis=1).astype(jnp.bfloat16)
        gather, scatter = halves[0]
        xj = gather(x.astype(jnp.bfloat16), src)
        msg = _edge_call(edge_attr, xj, W2T, 0, _E)
        return scatter(msg, dst, zero_init).reshape(2, _N, _C)

    p0 = layer(feature, nn_W0, nn_b0)
    x1 = _combine_call(p0, feature, lin_W0.T, bias0.reshape(1, _C))
    p1 = layer(x1, nn_W1, nn_b1)
    out = _combine_final_call(p1, x1, lin_W1.T, bias1.reshape(1, _C),
                              W_out.T, b_out.reshape(1, _C))
    return out


# speedup vs baseline: 1.0767x; 1.0767x over previous
import functools

import jax
import jax.numpy as jnp
from jax import lax
from jax.experimental import pallas as pl
from jax.experimental.pallas import tpu as pltpu
from jax.experimental.pallas import tpu_sc as plsc

_N = 10000
_E = 160000
_C = 64

_NC = 2
_NS = 16
_NW = _NC * _NS
_EPW = _E // _NW
_CH = 128
_NFULL = _EPW // _CH
_TAIL = _EPW - _NFULL * _CH
_RPT = _N // _NS

_HALVES = ((0, _E),)


def _make_gather_body(base, epw):
    nfull, tail = epw // _CH, epw % _CH
    assert nfull % 2 == 1 and tail > 0

    def body(x_hbm, idx_hbm, out_hbm, idx_all, rows0, rows1, rows_t,
             sem0, sem1):
        c = lax.axis_index("c")
        s = lax.axis_index("s")
        wid = s * _NC + c
        wbase = base + wid * epw
        pltpu.sync_copy(idx_hbm.at[pl.ds(pl.multiple_of(wbase, 8), epw)],
                        idx_all)

        def gst(j, buf, sem):
            pltpu.async_copy(x_hbm.at[idx_all.at[pl.ds(j * _CH, _CH)]],
                             buf, sem)

        def gwt(buf, sem):
            pltpu.make_async_copy(
                x_hbm.at[idx_all.at[pl.ds(0, _CH)]], buf, sem).wait()

        def wb(j, buf):
            pltpu.sync_copy(
                buf, out_hbm.at[pl.ds(pl.multiple_of(wbase + j * _CH, 8),
                                      _CH)])

        gst(0, rows0, sem0)

        def pair(p, carry):
            j = 2 * p
            gst(j + 1, rows1, sem1)
            gwt(rows0, sem0)
            wb(j, rows0)
            gst(j + 2, rows0, sem0)
            gwt(rows1, sem1)
            wb(j + 1, rows1)
            return carry

        lax.fori_loop(0, nfull // 2, pair, 0)
        gwt(rows0, sem0)
        wb(nfull - 1, rows0)
        offt = pl.multiple_of(wbase + nfull * _CH, 8)
        pltpu.async_copy(x_hbm.at[idx_all.at[pl.ds(nfull * _CH, tail)]],
                         rows_t, sem0).wait()
        pltpu.sync_copy(rows_t, out_hbm.at[pl.ds(offt, tail)])

    return body, tail


def _make_scatter_body(base, epw):
    nfull, tail = epw // _CH, epw % _CH
    assert nfull % 2 == 1 and tail > 0

    def body(msg_hbm, dst_hbm, zero_hbm, parts_hbm, idx0, idx1, rows0, rows1,
             idx_t, rows_t, agg_sh, si0, si1, sm0, sm1):
        c = lax.axis_index("c")
        s = lax.axis_index("s")
        wid = s * _NC + c
        wbase = base + wid * epw

        def lst(j, ib, rb, si, sm):
            off = pl.multiple_of(wbase + j * _CH, 8)
            pltpu.async_copy(dst_hbm.at[pl.ds(off, _CH)], ib, si)
            pltpu.async_copy(msg_hbm.at[pl.ds(off, _CH)], rb, sm)

        def lwt(ib, rb, si, sm):
            off0 = pl.multiple_of(wbase, 8)
            pltpu.make_async_copy(dst_hbm.at[pl.ds(off0, _CH)], ib, si).wait()
            pltpu.make_async_copy(msg_hbm.at[pl.ds(off0, _CH)], rb, sm).wait()

        lst(0, idx0, rows0, si0, sm0)
        pltpu.sync_copy(zero_hbm, agg_sh.at[pl.ds(s * _RPT, _RPT)])
        plsc.subcore_barrier()

        def pair(p, carry):
            j = 2 * p
            lst(j + 1, idx1, rows1, si1, sm1)
            lwt(idx0, rows0, si0, sm0)
            pltpu.sync_copy(rows0, agg_sh.at[idx0], add=True)
            lst(j + 2, idx0, rows0, si0, sm0)
            lwt(idx1, rows1, si1, sm1)
            pltpu.sync_copy(rows1, agg_sh.at[idx1], add=True)
            return carry

        lax.fori_loop(0, nfull // 2, pair, 0)
        lwt(idx0, rows0, si0, sm0)
        pltpu.sync_copy(rows0, agg_sh.at[idx0], add=True)
        offt = pl.multiple_of(wbase + nfull * _CH, 8)
        pltpu.sync_copy(dst_hbm.at[pl.ds(offt, tail)], idx_t)
        pltpu.sync_copy(msg_hbm.at[pl.ds(offt, tail)], rows_t)
        pltpu.sync_copy(rows_t, agg_sh.at[idx_t], add=True)
        plsc.subcore_barrier()
        pltpu.sync_copy(agg_sh.at[pl.ds(s * _RPT, _RPT)],
                        parts_hbm.at[pl.ds(c * _N + s * _RPT, _RPT)])

    return body, tail


@functools.lru_cache(maxsize=None)
def _sc_kernels():
    mesh = plsc.VectorSubcoreMesh(core_axis_name="c", subcore_axis_name="s")
    params = pltpu.CompilerParams(use_tc_tiling_on_sc=False)
    kernels = []
    for base, n_edges in _HALVES:
        epw = n_edges // _NW
        gbody, gtail = _make_gather_body(base, epw)
        gather = pl.kernel(
            gbody,
            mesh=mesh,
            compiler_params=params,
            out_type=jax.ShapeDtypeStruct((_E, _C), jnp.float32),
            scratch_types=[
                pltpu.VMEM((epw,), jnp.int32),
                pltpu.VMEM((_CH, _C), jnp.float32),
                pltpu.VMEM((_CH, _C), jnp.float32),
                pltpu.VMEM((max(gtail, 8), _C), jnp.float32),
                pltpu.SemaphoreType.DMA,
                pltpu.SemaphoreType.DMA,
            ],
        )
        sbody, stail = _make_scatter_body(base, epw)
        scatter = pl.kernel(
            sbody,
            mesh=mesh,
            compiler_params=params,
            out_type=jax.ShapeDtypeStruct((2 * _N, _C), jnp.float32),
            scratch_types=[
                pltpu.VMEM((_CH,), jnp.int32),
                pltpu.VMEM((_CH,), jnp.int32),
                pltpu.VMEM((_CH, _C), jnp.float32),
                pltpu.VMEM((_CH, _C), jnp.float32),
                pltpu.VMEM((max(stail, 8),), jnp.int32),
                pltpu.VMEM((max(stail, 8), _C), jnp.float32),
                pltpu.VMEM_SHARED((_N, _C), jnp.float32),
                pltpu.SemaphoreType.DMA,
                pltpu.SemaphoreType.DMA,
                pltpu.SemaphoreType.DMA,
                pltpu.SemaphoreType.DMA,
            ],
        )
        kernels.append((gather, scatter))
    return kernels


_BE = 2048
_BH = 256


def _edge_half(ea, xj, W2T):
    eaT = ea.T
    xjT = xj.T
    rep = jnp.broadcast_to(xjT[:, None, :],
                           (_C, _C, _BH)).reshape(_C * _C, _BH)
    eaT_t = jnp.tile(eaT, (_C, 1))
    u = (rep * eaT_t).astype(jnp.bfloat16)
    u_full = jnp.concatenate([u, xjT.astype(jnp.bfloat16)], axis=0)
    msgT = jnp.dot(W2T, u_full,
                   preferred_element_type=jnp.float32)
    return msgT.T


def _edge_body(ea_ref, xj_ref, W2T_ref, msg_ref):
    W2T = W2T_ref[...]
    for h in range(_BE // _BH):
        sl = pl.ds(h * _BH, _BH)
        msg_ref[sl, :] = _edge_half(ea_ref[sl, :], xj_ref[sl, :], W2T)


def _edge_call(ea, xj, W2T_bf16, blk0, n_edges):
    bmap = lambda i, b=blk0: (i + b, 0)
    return pl.pallas_call(
        _edge_body,
        grid=(pl.cdiv(n_edges, _BE),),
        in_specs=[
            pl.BlockSpec((_BE, _C), bmap),
            pl.BlockSpec((_BE, _C), bmap),
            pl.BlockSpec((_C, _C * _C + _C), lambda i: (0, 0)),
        ],
        out_specs=pl.BlockSpec((_BE, _C), bmap),
        out_shape=jax.ShapeDtypeStruct((_E, _C), jnp.float32),
    )(ea, xj, W2T_bf16)


_BN = 2000


def _combine_body(p_ref, x_ref, lwT_ref, b_ref, o_ref):
    s_ = (p_ref[0] + p_ref[1] + b_ref[...]
          + jnp.dot(x_ref[...], lwT_ref[...], preferred_element_type=jnp.float32))
    o_ref[...] = jnp.maximum(s_, 0.0)


def _combine_call(parts, x, lwT, b_row):
    return pl.pallas_call(
        _combine_body,
        grid=(_N // _BN,),
        in_specs=[
            pl.BlockSpec((2, _BN, _C), lambda i: (0, i, 0)),
            pl.BlockSpec((_BN, _C), lambda i: (i, 0)),
            pl.BlockSpec((_C, _C), lambda i: (0, 0)),
            pl.BlockSpec((1, _C), lambda i: (0, 0)),
        ],
        out_specs=pl.BlockSpec((_BN, _C), lambda i: (i, 0)),
        out_shape=jax.ShapeDtypeStruct((_N, _C), jnp.float32),
    )(parts, x, lwT, b_row)


def _combine_final_body(p_ref, x_ref, lwT_ref, b_ref, woT_ref, bo_ref, o_ref):
    s_ = (p_ref[0] + p_ref[1] + b_ref[...]
          + jnp.dot(x_ref[...], lwT_ref[...], preferred_element_type=jnp.float32))
    h = jnp.maximum(s_, 0.0)
    o_ref[...] = jnp.dot(h, woT_ref[...],
                         preferred_element_type=jnp.float32) + bo_ref[...]


def _combine_final_call(parts, x, lwT, b_row, woT, bo_row):
    return pl.pallas_call(
        _combine_final_body,
        grid=(_N // _BN,),
        in_specs=[
            pl.BlockSpec((2, _BN, _C), lambda i: (0, i, 0)),
            pl.BlockSpec((_BN, _C), lambda i: (i, 0)),
            pl.BlockSpec((_C, _C), lambda i: (0, 0)),
            pl.BlockSpec((1, _C), lambda i: (0, 0)),
            pl.BlockSpec((_C, _C), lambda i: (0, 0)),
            pl.BlockSpec((1, _C), lambda i: (0, 0)),
        ],
        out_specs=pl.BlockSpec((_BN, _C), lambda i: (i, 0)),
        out_shape=jax.ShapeDtypeStruct((_N, _C), jnp.float32),
    )(parts, x, lwT, b_row, woT, bo_row)


def kernel(feature, edge_index, edge_attr, nn_W0, nn_b0, lin_W0, bias0,
           nn_W1, nn_b1, lin_W1, bias1, W_out, b_out):
    src = edge_index[0]
    dst = edge_index[1]
    zero_init = jnp.zeros((_RPT, _C), jnp.float32)
    halves = _sc_kernels()

    def layer(x, nn_W, nn_b):
        W2T = nn_W.reshape(_C, _C, _C).transpose(1, 0, 2).reshape(_C, _C * _C)
        nbmT = nn_b.reshape(_C, _C).T
        W2T = jnp.concatenate([W2T, nbmT], axis=1).astype(jnp.bfloat16)
        gather, scatter = halves[0]
        xj = gather(x, src)
        msg = _edge_call(edge_attr, xj, W2T, 0, _E)
        return scatter(msg, dst, zero_init).reshape(2, _N, _C)

    p0 = layer(feature, nn_W0, nn_b0)
    x1 = _combine_call(p0, feature, lin_W0.T, bias0.reshape(1, _C))
    p1 = layer(x1, nn_W1, nn_b1)
    out = _combine_final_call(p1, x1, lin_W1.T, bias1.reshape(1, _C),
                              W_out.T, b_out.reshape(1, _C))
    return out


# speedup vs baseline: 1.2825x; 1.1911x over previous
import functools

import jax
import jax.numpy as jnp
from jax import lax
from jax.experimental import pallas as pl
from jax.experimental.pallas import tpu as pltpu
from jax.experimental.pallas import tpu_sc as plsc

_N = 10000
_E = 160000
_C = 64

_NC = 2
_NS = 16
_NW = _NC * _NS
_EPW = _E // _NW
_CH = 128
_NFULL = _EPW // _CH
_TAIL = _EPW - _NFULL * _CH
_RPT = _N // _NS
_W = 128

_HALVES = ((0, _E),)


def _make_gather_body(base, epw):
    nfull, tail = epw // _CH, epw % _CH
    assert nfull % 2 == 1 and tail > 0

    def body(x_hbm, idx_hbm, out_hbm, idx_all, rows0, rows1, rows_t,
             sem0, sem1):
        c = lax.axis_index("c")
        s = lax.axis_index("s")
        wid = s * _NC + c
        wbase = base + wid * epw
        pltpu.sync_copy(idx_hbm.at[pl.ds(pl.multiple_of(wbase, 8), epw)],
                        idx_all)

        def gst(j, buf, sem):
            pltpu.async_copy(x_hbm.at[idx_all.at[pl.ds(j * _CH, _CH)]],
                             buf, sem)

        def gwt(buf, sem):
            pltpu.make_async_copy(
                x_hbm.at[idx_all.at[pl.ds(0, _CH)]], buf, sem).wait()

        def wb(j, buf):
            pltpu.sync_copy(
                buf, out_hbm.at[pl.ds(pl.multiple_of(wbase + j * _CH, 8),
                                      _CH)])

        gst(0, rows0, sem0)

        def pair(p, carry):
            j = 2 * p
            gst(j + 1, rows1, sem1)
            gwt(rows0, sem0)
            wb(j, rows0)
            gst(j + 2, rows0, sem0)
            gwt(rows1, sem1)
            wb(j + 1, rows1)
            return carry

        lax.fori_loop(0, nfull // 2, pair, 0)
        gwt(rows0, sem0)
        wb(nfull - 1, rows0)
        offt = pl.multiple_of(wbase + nfull * _CH, 8)
        pltpu.async_copy(x_hbm.at[idx_all.at[pl.ds(nfull * _CH, tail)]],
                         rows_t, sem0).wait()
        pltpu.sync_copy(rows_t, out_hbm.at[pl.ds(offt, tail)])

    return body, tail


def _make_scatter_body(base, epw):
    nfull, tail = epw // _CH, epw % _CH
    assert nfull % 2 == 1 and tail > 0

    def body(msg_hbm, dst_hbm, zero_hbm, parts_hbm, idx0, idx1, rows0, rows1,
             idx_t, rows_t, agg_sh, si0, si1, sm0, sm1):
        c = lax.axis_index("c")
        s = lax.axis_index("s")
        wid = s * _NC + c
        wbase = base + wid * epw

        def lst(j, ib, rb, si, sm):
            off = pl.multiple_of(wbase + j * _CH, 8)
            pltpu.async_copy(dst_hbm.at[pl.ds(off, _CH)], ib, si)
            pltpu.async_copy(msg_hbm.at[pl.ds(off, _CH)], rb, sm)

        def lwt(ib, rb, si, sm):
            off0 = pl.multiple_of(wbase, 8)
            pltpu.make_async_copy(dst_hbm.at[pl.ds(off0, _CH)], ib, si).wait()
            pltpu.make_async_copy(msg_hbm.at[pl.ds(off0, _CH)], rb, sm).wait()

        lst(0, idx0, rows0, si0, sm0)
        r0 = pl.multiple_of(s * 632, 8)
        pltpu.sync_copy(zero_hbm.at[pl.ds(0, 520)], agg_sh.at[pl.ds(r0, 520)])

        @pl.when(s < _NS - 1)
        def _init_rest():
            pltpu.sync_copy(zero_hbm.at[pl.ds(520, 112)],
                            agg_sh.at[pl.ds(pl.multiple_of(r0 + 520, 8), 112)])

        plsc.subcore_barrier()

        def pair(p, carry):
            j = 2 * p
            lst(j + 1, idx1, rows1, si1, sm1)
            lwt(idx0, rows0, si0, sm0)
            pltpu.sync_copy(rows0, agg_sh.at[idx0], add=True)
            lst(j + 2, idx0, rows0, si0, sm0)
            lwt(idx1, rows1, si1, sm1)
            pltpu.sync_copy(rows1, agg_sh.at[idx1], add=True)
            return carry

        lax.fori_loop(0, nfull // 2, pair, 0)
        lwt(idx0, rows0, si0, sm0)
        pltpu.sync_copy(rows0, agg_sh.at[idx0], add=True)
        offt = pl.multiple_of(wbase + nfull * _CH, 8)
        pltpu.sync_copy(dst_hbm.at[pl.ds(offt, tail)], idx_t)
        pltpu.sync_copy(msg_hbm.at[pl.ds(offt, tail)], rows_t)
        pltpu.sync_copy(rows_t, agg_sh.at[idx_t], add=True)
        plsc.subcore_barrier()
        ro = pl.multiple_of(s * 632, 8)
        po = pl.multiple_of(c * _N + s * 632, 8)
        pltpu.sync_copy(agg_sh.at[pl.ds(ro, 520)],
                        parts_hbm.at[pl.ds(po, 520)])

        @pl.when(s < _NS - 1)
        def _out_rest():
            pltpu.sync_copy(
                agg_sh.at[pl.ds(pl.multiple_of(ro + 520, 8), 112)],
                parts_hbm.at[pl.ds(pl.multiple_of(po + 520, 8), 112)])

    return body, tail


@functools.lru_cache(maxsize=None)
def _sc_kernels():
    mesh = plsc.VectorSubcoreMesh(core_axis_name="c", subcore_axis_name="s")
    params = pltpu.CompilerParams(use_tc_tiling_on_sc=True)
    kernels = []
    for base, n_edges in _HALVES:
        epw = n_edges // _NW
        gbody, gtail = _make_gather_body(base, epw)
        gather = pl.kernel(
            gbody,
            mesh=mesh,
            compiler_params=params,
            out_type=jax.ShapeDtypeStruct((_E, _W), jnp.float32),
            scratch_types=[
                pltpu.VMEM((epw,), jnp.int32),
                pltpu.VMEM((_CH, _W), jnp.float32),
                pltpu.VMEM((_CH, _W), jnp.float32),
                pltpu.VMEM((max(gtail, 8), _W), jnp.float32),
                pltpu.SemaphoreType.DMA,
                pltpu.SemaphoreType.DMA,
            ],
        )
        sbody, stail = _make_scatter_body(base, epw)
        scatter = pl.kernel(
            sbody,
            mesh=mesh,
            compiler_params=params,
            out_type=jax.ShapeDtypeStruct((2 * _N, _W), jnp.float32),
            scratch_types=[
                pltpu.VMEM((_CH,), jnp.int32),
                pltpu.VMEM((_CH,), jnp.int32),
                pltpu.VMEM((_CH, _W), jnp.float32),
                pltpu.VMEM((_CH, _W), jnp.float32),
                pltpu.VMEM((max(stail, 8),), jnp.int32),
                pltpu.VMEM((max(stail, 8), _W), jnp.float32),
                pltpu.VMEM_SHARED((_N, _W), jnp.float32),
                pltpu.SemaphoreType.DMA,
                pltpu.SemaphoreType.DMA,
                pltpu.SemaphoreType.DMA,
                pltpu.SemaphoreType.DMA,
            ],
        )
        kernels.append((gather, scatter))
    return kernels


_BE = 2048
_BH = 256


def _edge_half(ea, xj, W2T):
    eaT = ea.T
    xjT = xj.T
    rep = jnp.broadcast_to(xjT[:, None, :],
                           (_C, _C, _BH)).reshape(_C * _C, _BH)
    eaT_t = jnp.tile(eaT, (_C, 1))
    u = (rep * eaT_t).astype(jnp.bfloat16)
    u_full = jnp.concatenate([u, xjT.astype(jnp.bfloat16)], axis=0)
    msgT = jnp.dot(W2T, u_full,
                   preferred_element_type=jnp.float32)
    return msgT.T


def _edge_body(ea_ref, xj_ref, W2T_ref, msg_ref):
    W2T = W2T_ref[...]
    zer = jnp.zeros((_BH, _W - _C), jnp.float32)
    for h in range(_BE // _BH):
        sl = pl.ds(h * _BH, _BH)
        m = _edge_half(ea_ref[sl, :], xj_ref[sl, :][:, :_C], W2T)
        msg_ref[sl, :] = jnp.concatenate([m, zer], axis=1)


def _edge_call(ea, xj, W2T_bf16, blk0, n_edges):
    bmap = lambda i, b=blk0: (i + b, 0)
    return pl.pallas_call(
        _edge_body,
        grid=(pl.cdiv(n_edges, _BE),),
        in_specs=[
            pl.BlockSpec((_BE, _C), bmap),
            pl.BlockSpec((_BE, _W), bmap),
            pl.BlockSpec((_C, _C * _C + _C), lambda i: (0, 0)),
        ],
        out_specs=pl.BlockSpec((_BE, _W), bmap),
        out_shape=jax.ShapeDtypeStruct((_E, _W), jnp.float32),
    )(ea, xj, W2T_bf16)


_BN = 2000


def _combine_body(p_ref, x_ref, lwT_ref, b_ref, o_ref):
    s_ = (p_ref[0][:, :_C] + p_ref[1][:, :_C] + b_ref[...]
          + jnp.dot(x_ref[...][:, :_C], lwT_ref[...],
                    preferred_element_type=jnp.float32))
    h = jnp.maximum(s_, 0.0)
    o_ref[...] = jnp.concatenate(
        [h, jnp.zeros((_BN, _W - _C), jnp.float32)], axis=1)


def _combine_call(parts, x, lwT, b_row):
    return pl.pallas_call(
        _combine_body,
        grid=(_N // _BN,),
        in_specs=[
            pl.BlockSpec((2, _BN, _W), lambda i: (0, i, 0)),
            pl.BlockSpec((_BN, _W), lambda i: (i, 0)),
            pl.BlockSpec((_C, _C), lambda i: (0, 0)),
            pl.BlockSpec((1, _C), lambda i: (0, 0)),
        ],
        out_specs=pl.BlockSpec((_BN, _W), lambda i: (i, 0)),
        out_shape=jax.ShapeDtypeStruct((_N, _W), jnp.float32),
    )(parts, x, lwT, b_row)


def _combine_final_body(p_ref, x_ref, lwT_ref, b_ref, woT_ref, bo_ref, o_ref):
    s_ = (p_ref[0][:, :_C] + p_ref[1][:, :_C] + b_ref[...]
          + jnp.dot(x_ref[...][:, :_C], lwT_ref[...],
                    preferred_element_type=jnp.float32))
    h = jnp.maximum(s_, 0.0)
    o_ref[...] = jnp.dot(h, woT_ref[...],
                         preferred_element_type=jnp.float32) + bo_ref[...]


def _combine_final_call(parts, x, lwT, b_row, woT, bo_row):
    return pl.pallas_call(
        _combine_final_body,
        grid=(_N // _BN,),
        in_specs=[
            pl.BlockSpec((2, _BN, _W), lambda i: (0, i, 0)),
            pl.BlockSpec((_BN, _W), lambda i: (i, 0)),
            pl.BlockSpec((_C, _C), lambda i: (0, 0)),
            pl.BlockSpec((1, _C), lambda i: (0, 0)),
            pl.BlockSpec((_C, _C), lambda i: (0, 0)),
            pl.BlockSpec((1, _C), lambda i: (0, 0)),
        ],
        out_specs=pl.BlockSpec((_BN, _C), lambda i: (i, 0)),
        out_shape=jax.ShapeDtypeStruct((_N, _C), jnp.float32),
    )(parts, x, lwT, b_row, woT, bo_row)


def kernel(feature, edge_index, edge_attr, nn_W0, nn_b0, lin_W0, bias0,
           nn_W1, nn_b1, lin_W1, bias1, W_out, b_out):
    src = edge_index[0]
    dst = edge_index[1]
    zero_init = jnp.zeros((632, _W), jnp.float32)
    halves = _sc_kernels()

    def layer(x, nn_W, nn_b):
        W2T = nn_W.reshape(_C, _C, _C).transpose(1, 0, 2).reshape(_C, _C * _C)
        nbmT = nn_b.reshape(_C, _C).T
        W2T = jnp.concatenate([W2T, nbmT], axis=1).astype(jnp.bfloat16)
        gather, scatter = halves[0]
        xj = gather(x, src)
        msg = _edge_call(edge_attr, xj, W2T, 0, _E)
        return scatter(msg, dst, zero_init).reshape(2, _N, _W)

    feat_pad = jnp.concatenate(
        [feature, jnp.zeros((_N, _W - _C), jnp.float32)], axis=1)
    p0 = layer(feat_pad, nn_W0, nn_b0)
    x1 = _combine_call(p0, feat_pad, lin_W0.T, bias0.reshape(1, _C))
    p1 = layer(x1, nn_W1, nn_b1)
    out = _combine_final_call(p1, x1, lin_W1.T, bias1.reshape(1, _C),
                              W_out.T, b_out.reshape(1, _C))
    return out


# speedup vs baseline: 1.3167x; 1.0267x over previous
import functools

import jax
import jax.numpy as jnp
from jax import lax
from jax.experimental import pallas as pl
from jax.experimental.pallas import tpu as pltpu
from jax.experimental.pallas import tpu_sc as plsc

_N = 10000
_E = 160000
_C = 64

_NC = 2
_NS = 16
_NW = _NC * _NS
_EPW = _E // _NW
_CH = 128
_NFULL = _EPW // _CH
_TAIL = _EPW - _NFULL * _CH
_RPT = _N // _NS
_W = 128

_HALVES = ((0, _E),)


def _make_gather_body(base, epw):
    nfull, tail = epw // _CH, epw % _CH
    assert nfull % 2 == 1 and tail > 0

    def body(x_hbm, idx_hbm, out_hbm, idx_all, rows0, rows1, rows_t,
             sem0, sem1):
        c = lax.axis_index("c")
        s = lax.axis_index("s")
        wid = s * _NC + c
        wbase = base + wid * epw
        pltpu.sync_copy(idx_hbm.at[pl.ds(pl.multiple_of(wbase, 8), epw)],
                        idx_all)

        def gst(j, buf, sem):
            pltpu.async_copy(x_hbm.at[idx_all.at[pl.ds(j * _CH, _CH)]],
                             buf, sem)

        def gwt(buf, sem):
            pltpu.make_async_copy(
                x_hbm.at[idx_all.at[pl.ds(0, _CH)]], buf, sem).wait()

        def wb(j, buf):
            pltpu.sync_copy(
                buf, out_hbm.at[pl.ds(pl.multiple_of(wbase + j * _CH, 8),
                                      _CH)])

        gst(0, rows0, sem0)

        def pair(p, carry):
            j = 2 * p
            gst(j + 1, rows1, sem1)
            gwt(rows0, sem0)
            wb(j, rows0)
            gst(j + 2, rows0, sem0)
            gwt(rows1, sem1)
            wb(j + 1, rows1)
            return carry

        lax.fori_loop(0, nfull // 2, pair, 0)
        gwt(rows0, sem0)
        wb(nfull - 1, rows0)
        offt = pl.multiple_of(wbase + nfull * _CH, 8)
        pltpu.async_copy(x_hbm.at[idx_all.at[pl.ds(nfull * _CH, tail)]],
                         rows_t, sem0).wait()
        pltpu.sync_copy(rows_t, out_hbm.at[pl.ds(offt, tail)])

    return body, tail


def _make_scatter_body(base, epw):
    nfull, tail = epw // _CH, epw % _CH
    assert nfull % 2 == 1 and tail > 0

    def body(msg_hbm, dst_hbm, zero_hbm, parts_hbm, idx0, idx1, rows0, rows1,
             idx_t, rows_t, agg_sh, si0, si1, sm0, sm1):
        c = lax.axis_index("c")
        s = lax.axis_index("s")
        wid = s * _NC + c
        wbase = base + wid * epw

        def lst(j, ib, rb, si, sm):
            off = pl.multiple_of(wbase + j * _CH, 8)
            pltpu.async_copy(dst_hbm.at[pl.ds(off, _CH)], ib, si)
            pltpu.async_copy(msg_hbm.at[pl.ds(off, _CH)], rb, sm)

        def lwt(ib, rb, si, sm):
            off0 = pl.multiple_of(wbase, 8)
            pltpu.make_async_copy(dst_hbm.at[pl.ds(off0, _CH)], ib, si).wait()
            pltpu.make_async_copy(msg_hbm.at[pl.ds(off0, _CH)], rb, sm).wait()

        lst(0, idx0, rows0, si0, sm0)
        r0 = pl.multiple_of(s * 632, 8)
        pltpu.sync_copy(zero_hbm.at[pl.ds(0, 520)], agg_sh.at[pl.ds(r0, 520)])

        @pl.when(s < _NS - 1)
        def _init_rest():
            pltpu.sync_copy(zero_hbm.at[pl.ds(520, 112)],
                            agg_sh.at[pl.ds(pl.multiple_of(r0 + 520, 8), 112)])

        plsc.subcore_barrier()

        def pair(p, carry):
            j = 2 * p
            lst(j + 1, idx1, rows1, si1, sm1)
            lwt(idx0, rows0, si0, sm0)
            pltpu.sync_copy(rows0, agg_sh.at[idx0], add=True)
            lst(j + 2, idx0, rows0, si0, sm0)
            lwt(idx1, rows1, si1, sm1)
            pltpu.sync_copy(rows1, agg_sh.at[idx1], add=True)
            return carry

        lax.fori_loop(0, nfull // 2, pair, 0)
        lwt(idx0, rows0, si0, sm0)
        pltpu.sync_copy(rows0, agg_sh.at[idx0], add=True)
        offt = pl.multiple_of(wbase + nfull * _CH, 8)
        pltpu.sync_copy(dst_hbm.at[pl.ds(offt, tail)], idx_t)
        pltpu.sync_copy(msg_hbm.at[pl.ds(offt, tail)], rows_t)
        pltpu.sync_copy(rows_t, agg_sh.at[idx_t], add=True)
        plsc.subcore_barrier()
        ro = pl.multiple_of(s * 632, 8)
        po = pl.multiple_of(c * _N + s * 632, 8)
        pltpu.sync_copy(agg_sh.at[pl.ds(ro, 520)],
                        parts_hbm.at[pl.ds(po, 520)])

        @pl.when(s < _NS - 1)
        def _out_rest():
            pltpu.sync_copy(
                agg_sh.at[pl.ds(pl.multiple_of(ro + 520, 8), 112)],
                parts_hbm.at[pl.ds(pl.multiple_of(po + 520, 8), 112)])

    return body, tail


@functools.lru_cache(maxsize=None)
def _sc_kernels():
    mesh = plsc.VectorSubcoreMesh(core_axis_name="c", subcore_axis_name="s")
    params = pltpu.CompilerParams(use_tc_tiling_on_sc=True)
    kernels = []
    for base, n_edges in _HALVES:
        epw = n_edges // _NW
        gbody, gtail = _make_gather_body(base, epw)
        gather = pl.kernel(
            gbody,
            mesh=mesh,
            compiler_params=params,
            out_type=jax.ShapeDtypeStruct((_E, _W), jnp.float32),
            scratch_types=[
                pltpu.VMEM((epw,), jnp.int32),
                pltpu.VMEM((_CH, _W), jnp.float32),
                pltpu.VMEM((_CH, _W), jnp.float32),
                pltpu.VMEM((max(gtail, 8), _W), jnp.float32),
                pltpu.SemaphoreType.DMA,
                pltpu.SemaphoreType.DMA,
            ],
        )
        sbody, stail = _make_scatter_body(base, epw)
        scatter = pl.kernel(
            sbody,
            mesh=mesh,
            compiler_params=params,
            out_type=jax.ShapeDtypeStruct((2 * _N, _W), jnp.float32),
            scratch_types=[
                pltpu.VMEM((_CH,), jnp.int32),
                pltpu.VMEM((_CH,), jnp.int32),
                pltpu.VMEM((_CH, _W), jnp.float32),
                pltpu.VMEM((_CH, _W), jnp.float32),
                pltpu.VMEM((max(stail, 8),), jnp.int32),
                pltpu.VMEM((max(stail, 8), _W), jnp.float32),
                pltpu.VMEM_SHARED((_N, _W), jnp.float32),
                pltpu.SemaphoreType.DMA,
                pltpu.SemaphoreType.DMA,
                pltpu.SemaphoreType.DMA,
                pltpu.SemaphoreType.DMA,
            ],
        )
        kernels.append((gather, scatter))
    return kernels


_BE = 2048
_BH = 256


def _edge_half(ea, xj, W2T):
    eaT = ea.T.astype(jnp.float32)
    xjT = xj.T
    rep = jnp.broadcast_to(xjT[:, None, :],
                           (_C, _C, _BH)).reshape(_C * _C, _BH)
    eaT_t = jnp.tile(eaT, (_C, 1))
    u = (rep * eaT_t).astype(jnp.bfloat16)
    u_full = jnp.concatenate([u, xjT.astype(jnp.bfloat16)], axis=0)
    msgT = jnp.dot(W2T, u_full,
                   preferred_element_type=jnp.float32)
    return msgT.T


def _edge_body(ea_ref, xj_ref, W2T_ref, msg_ref):
    W2T = W2T_ref[...]
    zer = jnp.zeros((_BH, _W - _C), jnp.float32)
    for h in range(_BE // _BH):
        sl = pl.ds(h * _BH, _BH)
        m = _edge_half(ea_ref[sl, :], xj_ref[sl, :][:, :_C], W2T)
        msg_ref[sl, :] = jnp.concatenate([m, zer], axis=1)


def _edge_call(ea, xj, W2T_bf16, blk0, n_edges):
    bmap = lambda i, b=blk0: (i + b, 0)
    return pl.pallas_call(
        _edge_body,
        grid=(pl.cdiv(n_edges, _BE),),
        in_specs=[
            pl.BlockSpec((_BE, _C), bmap),
            pl.BlockSpec((_BE, _W), bmap),
            pl.BlockSpec((_C, _C * _C + _C), lambda i: (0, 0)),
        ],
        out_specs=pl.BlockSpec((_BE, _W), bmap),
        out_shape=jax.ShapeDtypeStruct((_E, _W), jnp.float32),
    )(ea, xj, W2T_bf16)


_BN = 2000


def _combine_body(p_ref, x_ref, lwT_ref, b_ref, o_ref):
    s_ = (p_ref[0][:, :_C] + p_ref[1][:, :_C] + b_ref[...]
          + jnp.dot(x_ref[...][:, :_C], lwT_ref[...],
                    preferred_element_type=jnp.float32))
    h = jnp.maximum(s_, 0.0)
    o_ref[...] = jnp.concatenate(
        [h, jnp.zeros((_BN, _W - _C), jnp.float32)], axis=1)


def _combine_call(parts, x, lwT, b_row):
    return pl.pallas_call(
        _combine_body,
        grid=(_N // _BN,),
        in_specs=[
            pl.BlockSpec((2, _BN, _W), lambda i: (0, i, 0)),
            pl.BlockSpec((_BN, _W), lambda i: (i, 0)),
            pl.BlockSpec((_C, _C), lambda i: (0, 0)),
            pl.BlockSpec((1, _C), lambda i: (0, 0)),
        ],
        out_specs=pl.BlockSpec((_BN, _W), lambda i: (i, 0)),
        out_shape=jax.ShapeDtypeStruct((_N, _W), jnp.float32),
    )(parts, x, lwT, b_row)


def _combine_final_body(p_ref, x_ref, lwT_ref, b_ref, woT_ref, bo_ref, o_ref):
    s_ = (p_ref[0][:, :_C] + p_ref[1][:, :_C] + b_ref[...]
          + jnp.dot(x_ref[...][:, :_C], lwT_ref[...],
                    preferred_element_type=jnp.float32))
    h = jnp.maximum(s_, 0.0)
    o_ref[...] = jnp.dot(h, woT_ref[...],
                         preferred_element_type=jnp.float32) + bo_ref[...]


def _combine_final_call(parts, x, lwT, b_row, woT, bo_row):
    return pl.pallas_call(
        _combine_final_body,
        grid=(_N // _BN,),
        in_specs=[
            pl.BlockSpec((2, _BN, _W), lambda i: (0, i, 0)),
            pl.BlockSpec((_BN, _W), lambda i: (i, 0)),
            pl.BlockSpec((_C, _C), lambda i: (0, 0)),
            pl.BlockSpec((1, _C), lambda i: (0, 0)),
            pl.BlockSpec((_C, _C), lambda i: (0, 0)),
            pl.BlockSpec((1, _C), lambda i: (0, 0)),
        ],
        out_specs=pl.BlockSpec((_BN, _C), lambda i: (i, 0)),
        out_shape=jax.ShapeDtypeStruct((_N, _C), jnp.float32),
    )(parts, x, lwT, b_row, woT, bo_row)


def kernel(feature, edge_index, edge_attr, nn_W0, nn_b0, lin_W0, bias0,
           nn_W1, nn_b1, lin_W1, bias1, W_out, b_out):
    src = edge_index[0]
    dst = edge_index[1]
    zero_init = jnp.zeros((632, _W), jnp.float32)
    halves = _sc_kernels()

    def layer(x, nn_W, nn_b):
        W2T = nn_W.reshape(_C, _C, _C).transpose(1, 0, 2).reshape(_C, _C * _C)
        nbmT = nn_b.reshape(_C, _C).T
        W2T = jnp.concatenate([W2T, nbmT], axis=1).astype(jnp.bfloat16)
        gather, scatter = halves[0]
        xj = gather(x, src)
        msg = _edge_call(ea_b, xj, W2T, 0, _E)
        return scatter(msg, dst, zero_init).reshape(2, _N, _W)

    ea_b = edge_attr.astype(jnp.bfloat16)
    feat_pad = jnp.concatenate(
        [feature, jnp.zeros((_N, _W - _C), jnp.float32)], axis=1)
    p0 = layer(feat_pad, nn_W0, nn_b0)
    x1 = _combine_call(p0, feat_pad, lin_W0.T, bias0.reshape(1, _C))
    p1 = layer(x1, nn_W1, nn_b1)
    out = _combine_final_call(p1, x1, lin_W1.T, bias1.reshape(1, _C),
                              W_out.T, b_out.reshape(1, _C))
    return out


# speedup vs baseline: 1.3760x; 1.0450x over previous
import functools

import jax
import jax.numpy as jnp
from jax import lax
from jax.experimental import pallas as pl
from jax.experimental.pallas import tpu as pltpu
from jax.experimental.pallas import tpu_sc as plsc

_N = 10000
_E = 160000
_C = 64

_NC = 2
_NS = 16
_NW = _NC * _NS
_EPW = _E // _NW
_CH = 128
_NFULL = _EPW // _CH
_TAIL = _EPW - _NFULL * _CH
_RPT = _N // _NS
_W = 128

_HALVES = ((0, _E),)


def _make_gather_body(base, epw):
    nfull, tail = epw // _CH, epw % _CH
    assert nfull % 2 == 1 and tail > 0

    def body(x_hbm, idx_hbm, out_hbm, idx_all, rows0, rows1, rows_t,
             sem0, sem1):
        c = lax.axis_index("c")
        s = lax.axis_index("s")
        wid = s * _NC + c
        wbase = base + wid * epw
        pltpu.sync_copy(idx_hbm.at[pl.ds(pl.multiple_of(wbase, 8), epw)],
                        idx_all)

        def gst(j, buf, sem):
            pltpu.async_copy(x_hbm.at[idx_all.at[pl.ds(j * _CH, _CH)]],
                             buf, sem)

        def gwt(buf, sem):
            pltpu.make_async_copy(
                x_hbm.at[idx_all.at[pl.ds(0, _CH)]], buf, sem).wait()

        def wb(j, buf):
            pltpu.sync_copy(
                buf, out_hbm.at[pl.ds(pl.multiple_of(wbase + j * _CH, 8),
                                      _CH)])

        gst(0, rows0, sem0)

        def pair(p, carry):
            j = 2 * p
            gst(j + 1, rows1, sem1)
            gwt(rows0, sem0)
            wb(j, rows0)
            gst(j + 2, rows0, sem0)
            gwt(rows1, sem1)
            wb(j + 1, rows1)
            return carry

        lax.fori_loop(0, nfull // 2, pair, 0)
        gwt(rows0, sem0)
        wb(nfull - 1, rows0)
        offt = pl.multiple_of(wbase + nfull * _CH, 8)
        pltpu.async_copy(x_hbm.at[idx_all.at[pl.ds(nfull * _CH, tail)]],
                         rows_t, sem0).wait()
        pltpu.sync_copy(rows_t, out_hbm.at[pl.ds(offt, tail)])

    return body, tail


def _make_scatter_body(base, epw):
    nfull, tail = epw // _CH, epw % _CH
    assert nfull % 2 == 1 and tail > 0

    def body(msg_hbm, dst_hbm, zero_hbm, parts_hbm, idx0, idx1, rows0, rows1,
             idx_t, rows_t, agg_sh, si0, si1, sm0, sm1):
        c = lax.axis_index("c")
        s = lax.axis_index("s")
        wid = s * _NC + c
        wbase = base + wid * epw

        def lst(j, ib, rb, si, sm):
            off = pl.multiple_of(wbase + j * _CH, 8)
            pltpu.async_copy(dst_hbm.at[pl.ds(off, _CH)], ib, si)
            pltpu.async_copy(msg_hbm.at[pl.ds(off, _CH)], rb, sm)

        def lwt(ib, rb, si, sm):
            off0 = pl.multiple_of(wbase, 8)
            pltpu.make_async_copy(dst_hbm.at[pl.ds(off0, _CH)], ib, si).wait()
            pltpu.make_async_copy(msg_hbm.at[pl.ds(off0, _CH)], rb, sm).wait()

        lst(0, idx0, rows0, si0, sm0)
        r0 = pl.multiple_of(s * 632, 8)
        pltpu.sync_copy(zero_hbm.at[pl.ds(0, 520)], agg_sh.at[pl.ds(r0, 520)])

        @pl.when(s < _NS - 1)
        def _init_rest():
            pltpu.sync_copy(zero_hbm.at[pl.ds(520, 112)],
                            agg_sh.at[pl.ds(pl.multiple_of(r0 + 520, 8), 112)])

        plsc.subcore_barrier()

        def pair(p, carry):
            j = 2 * p
            lst(j + 1, idx1, rows1, si1, sm1)
            lwt(idx0, rows0, si0, sm0)
            pltpu.sync_copy(rows0, agg_sh.at[idx0], add=True)
            lst(j + 2, idx0, rows0, si0, sm0)
            lwt(idx1, rows1, si1, sm1)
            pltpu.sync_copy(rows1, agg_sh.at[idx1], add=True)
            return carry

        lax.fori_loop(0, nfull // 2, pair, 0)
        lwt(idx0, rows0, si0, sm0)
        pltpu.sync_copy(rows0, agg_sh.at[idx0], add=True)
        offt = pl.multiple_of(wbase + nfull * _CH, 8)
        pltpu.sync_copy(dst_hbm.at[pl.ds(offt, tail)], idx_t)
        pltpu.sync_copy(msg_hbm.at[pl.ds(offt, tail)], rows_t)
        pltpu.sync_copy(rows_t, agg_sh.at[idx_t], add=True)
        plsc.subcore_barrier()
        ro = pl.multiple_of(s * 632, 8)
        po = pl.multiple_of(c * _N + s * 632, 8)
        pltpu.sync_copy(agg_sh.at[pl.ds(ro, 520)],
                        parts_hbm.at[pl.ds(po, 520)])

        @pl.when(s < _NS - 1)
        def _out_rest():
            pltpu.sync_copy(
                agg_sh.at[pl.ds(pl.multiple_of(ro + 520, 8), 112)],
                parts_hbm.at[pl.ds(pl.multiple_of(po + 520, 8), 112)])

    return body, tail


@functools.lru_cache(maxsize=None)
def _sc_kernels():
    mesh = plsc.VectorSubcoreMesh(core_axis_name="c", subcore_axis_name="s")
    params = pltpu.CompilerParams(use_tc_tiling_on_sc=True)
    kernels = []
    for base, n_edges in _HALVES:
        epw = n_edges // _NW
        gbody, gtail = _make_gather_body(base, epw)
        gather = pl.kernel(
            gbody,
            mesh=mesh,
            compiler_params=params,
            out_type=jax.ShapeDtypeStruct((_E, _W), jnp.float32),
            scratch_types=[
                pltpu.VMEM((epw,), jnp.int32),
                pltpu.VMEM((_CH, _W), jnp.float32),
                pltpu.VMEM((_CH, _W), jnp.float32),
                pltpu.VMEM((max(gtail, 8), _W), jnp.float32),
                pltpu.SemaphoreType.DMA,
                pltpu.SemaphoreType.DMA,
            ],
        )
        sbody, stail = _make_scatter_body(base, epw)
        scatter = pl.kernel(
            sbody,
            mesh=mesh,
            compiler_params=params,
            out_type=jax.ShapeDtypeStruct((2 * _N, _W), jnp.float32),
            scratch_types=[
                pltpu.VMEM((_CH,), jnp.int32),
                pltpu.VMEM((_CH,), jnp.int32),
                pltpu.VMEM((_CH, _W), jnp.float32),
                pltpu.VMEM((_CH, _W), jnp.float32),
                pltpu.VMEM((max(stail, 8),), jnp.int32),
                pltpu.VMEM((max(stail, 8), _W), jnp.float32),
                pltpu.VMEM_SHARED((_N, _W), jnp.float32),
                pltpu.SemaphoreType.DMA,
                pltpu.SemaphoreType.DMA,
                pltpu.SemaphoreType.DMA,
                pltpu.SemaphoreType.DMA,
            ],
        )
        kernels.append((gather, scatter))
    return kernels


_BE = 4096
_BH = 256


def _edge_half(ea, xj, W2T):
    eaT = ea.T.astype(jnp.float32)
    xjT = xj.T
    rep = jnp.broadcast_to(xjT[:, None, :],
                           (_C, _C, _BH)).reshape(_C * _C, _BH)
    eaT_t = jnp.tile(eaT, (_C, 1))
    u = (rep * eaT_t).astype(jnp.bfloat16)
    u_full = jnp.concatenate([u, xjT.astype(jnp.bfloat16)], axis=0)
    msgT = jnp.dot(W2T, u_full,
                   preferred_element_type=jnp.float32)
    return msgT.T


def _edge_body(ea_ref, xj_ref, W2T_ref, msg_ref):
    W2T = W2T_ref[...]
    zer = jnp.zeros((_BH, _W - _C), jnp.float32)
    for h in range(_BE // _BH):
        sl = pl.ds(h * _BH, _BH)
        m = _edge_half(ea_ref[sl, :], xj_ref[sl, :][:, :_C], W2T)
        msg_ref[sl, :] = jnp.concatenate([m, zer], axis=1)


def _edge_call(ea, xj, W2T_bf16, blk0, n_edges):
    bmap = lambda i, b=blk0: (i + b, 0)
    return pl.pallas_call(
        _edge_body,
        grid=(pl.cdiv(n_edges, _BE),),
        in_specs=[
            pl.BlockSpec((_BE, _C), bmap),
            pl.BlockSpec((_BE, _W), bmap),
            pl.BlockSpec((_C, _C * _C + _C), lambda i: (0, 0)),
        ],
        out_specs=pl.BlockSpec((_BE, _W), bmap),
        out_shape=jax.ShapeDtypeStruct((_E, _W), jnp.float32),
    )(ea, xj, W2T_bf16)


_BN = 2000


def _combine_body(p_ref, x_ref, lwT_ref, b_ref, o_ref):
    s_ = (p_ref[0][:, :_C] + p_ref[1][:, :_C] + b_ref[...]
          + jnp.dot(x_ref[...][:, :_C], lwT_ref[...],
                    preferred_element_type=jnp.float32))
    h = jnp.maximum(s_, 0.0)
    o_ref[...] = jnp.concatenate(
        [h, jnp.zeros((_BN, _W - _C), jnp.float32)], axis=1)


def _combine_call(parts, x, lwT, b_row):
    return pl.pallas_call(
        _combine_body,
        grid=(_N // _BN,),
        in_specs=[
            pl.BlockSpec((2, _BN, _W), lambda i: (0, i, 0)),
            pl.BlockSpec((_BN, _W), lambda i: (i, 0)),
            pl.BlockSpec((_C, _C), lambda i: (0, 0)),
            pl.BlockSpec((1, _C), lambda i: (0, 0)),
        ],
        out_specs=pl.BlockSpec((_BN, _W), lambda i: (i, 0)),
        out_shape=jax.ShapeDtypeStruct((_N, _W), jnp.float32),
    )(parts, x, lwT, b_row)


def _combine_final_body(p_ref, x_ref, lwT_ref, b_ref, woT_ref, bo_ref, o_ref):
    s_ = (p_ref[0][:, :_C] + p_ref[1][:, :_C] + b_ref[...]
          + jnp.dot(x_ref[...][:, :_C], lwT_ref[...],
                    preferred_element_type=jnp.float32))
    h = jnp.maximum(s_, 0.0)
    o_ref[...] = jnp.dot(h, woT_ref[...],
                         preferred_element_type=jnp.float32) + bo_ref[...]


def _combine_final_call(parts, x, lwT, b_row, woT, bo_row):
    return pl.pallas_call(
        _combine_final_body,
        grid=(_N // _BN,),
        in_specs=[
            pl.BlockSpec((2, _BN, _W), lambda i: (0, i, 0)),
            pl.BlockSpec((_BN, _W), lambda i: (i, 0)),
            pl.BlockSpec((_C, _C), lambda i: (0, 0)),
            pl.BlockSpec((1, _C), lambda i: (0, 0)),
            pl.BlockSpec((_C, _C), lambda i: (0, 0)),
            pl.BlockSpec((1, _C), lambda i: (0, 0)),
        ],
        out_specs=pl.BlockSpec((_BN, _C), lambda i: (i, 0)),
        out_shape=jax.ShapeDtypeStruct((_N, _C), jnp.float32),
    )(parts, x, lwT, b_row, woT, bo_row)


def kernel(feature, edge_index, edge_attr, nn_W0, nn_b0, lin_W0, bias0,
           nn_W1, nn_b1, lin_W1, bias1, W_out, b_out):
    src = edge_index[0]
    dst = edge_index[1]
    zero_init = jnp.zeros((632, _W), jnp.float32)
    halves = _sc_kernels()

    def layer(x, nn_W, nn_b):
        W2T = nn_W.reshape(_C, _C, _C).transpose(1, 0, 2).reshape(_C, _C * _C)
        nbmT = nn_b.reshape(_C, _C).T
        W2T = jnp.concatenate([W2T, nbmT], axis=1).astype(jnp.bfloat16)
        gather, scatter = halves[0]
        xj = gather(x, src)
        msg = _edge_call(ea_b, xj, W2T, 0, _E)
        return scatter(msg, dst, zero_init).reshape(2, _N, _W)

    ea_b = edge_attr.astype(jnp.bfloat16)
    feat_pad = jnp.concatenate(
        [feature, jnp.zeros((_N, _W - _C), jnp.float32)], axis=1)
    p0 = layer(feat_pad, nn_W0, nn_b0)
    x1 = _combine_call(p0, feat_pad, lin_W0.T, bias0.reshape(1, _C))
    p1 = layer(x1, nn_W1, nn_b1)
    out = _combine_final_call(p1, x1, lin_W1.T, bias1.reshape(1, _C),
                              W_out.T, b_out.reshape(1, _C))
    return out


# speedup vs baseline: 1.3978x; 1.0159x over previous
import functools

import jax
import jax.numpy as jnp
from jax import lax
from jax.experimental import pallas as pl
from jax.experimental.pallas import tpu as pltpu
from jax.experimental.pallas import tpu_sc as plsc

_N = 10000
_E = 160000
_C = 64

_NC = 2
_NS = 16
_NW = _NC * _NS
_EPW = _E // _NW
_CH = 128
_NFULL = _EPW // _CH
_TAIL = _EPW - _NFULL * _CH
_RPT = _N // _NS
_W = 128

_HALVES = ((0, _E),)


def _make_gather_body(base, epw):
    nfull, tail = epw // _CH, epw % _CH
    assert nfull % 2 == 1 and tail > 0

    def body(x_hbm, idx_hbm, out_hbm, idx_all, rows0, rows1, rows_t,
             sem0, sem1):
        c = lax.axis_index("c")
        s = lax.axis_index("s")
        wid = s * _NC + c
        wbase = base + wid * epw
        pltpu.sync_copy(idx_hbm.at[pl.ds(pl.multiple_of(wbase, 8), epw)],
                        idx_all)

        def gst(j, buf, sem):
            pltpu.async_copy(x_hbm.at[idx_all.at[pl.ds(j * _CH, _CH)]],
                             buf, sem)

        def gwt(buf, sem):
            pltpu.make_async_copy(
                x_hbm.at[idx_all.at[pl.ds(0, _CH)]], buf, sem).wait()

        def wb(j, buf):
            pltpu.sync_copy(
                buf, out_hbm.at[pl.ds(pl.multiple_of(wbase + j * _CH, 8),
                                      _CH)])

        gst(0, rows0, sem0)

        def pair(p, carry):
            j = 2 * p
            gst(j + 1, rows1, sem1)
            gwt(rows0, sem0)
            wb(j, rows0)
            gst(j + 2, rows0, sem0)
            gwt(rows1, sem1)
            wb(j + 1, rows1)
            return carry

        lax.fori_loop(0, nfull // 2, pair, 0)
        gwt(rows0, sem0)
        wb(nfull - 1, rows0)
        offt = pl.multiple_of(wbase + nfull * _CH, 8)
        pltpu.async_copy(x_hbm.at[idx_all.at[pl.ds(nfull * _CH, tail)]],
                         rows_t, sem0).wait()
        pltpu.sync_copy(rows_t, out_hbm.at[pl.ds(offt, tail)])

    return body, tail


def _make_scatter_body(base, epw):
    nfull, tail = epw // _CH, epw % _CH
    assert nfull % 2 == 1 and tail > 0

    def body(msg_hbm, dst_hbm, zero_hbm, parts_hbm, idx0, idx1, rows0, rows1,
             idx_t, rows_t, agg_sh, si0, si1, sm0, sm1):
        c = lax.axis_index("c")
        s = lax.axis_index("s")
        wid = s * _NC + c
        wbase = base + wid * epw

        def lst(j, ib, rb, si, sm):
            off = pl.multiple_of(wbase + j * _CH, 8)
            pltpu.async_copy(dst_hbm.at[pl.ds(off, _CH)], ib, si)
            pltpu.async_copy(msg_hbm.at[pl.ds(off, _CH)], rb, sm)

        def lwt(ib, rb, si, sm):
            off0 = pl.multiple_of(wbase, 8)
            pltpu.make_async_copy(dst_hbm.at[pl.ds(off0, _CH)], ib, si).wait()
            pltpu.make_async_copy(msg_hbm.at[pl.ds(off0, _CH)], rb, sm).wait()

        lst(0, idx0, rows0, si0, sm0)
        r0 = pl.multiple_of(s * 632, 8)
        pltpu.sync_copy(zero_hbm.at[pl.ds(0, 520)], agg_sh.at[pl.ds(r0, 520)])

        @pl.when(s < _NS - 1)
        def _init_rest():
            pltpu.sync_copy(zero_hbm.at[pl.ds(520, 112)],
                            agg_sh.at[pl.ds(pl.multiple_of(r0 + 520, 8), 112)])

        plsc.subcore_barrier()

        def pair(p, carry):
            j = 2 * p
            lst(j + 1, idx1, rows1, si1, sm1)
            lwt(idx0, rows0, si0, sm0)
            pltpu.sync_copy(rows0, agg_sh.at[idx0], add=True)
            lst(j + 2, idx0, rows0, si0, sm0)
            lwt(idx1, rows1, si1, sm1)
            pltpu.sync_copy(rows1, agg_sh.at[idx1], add=True)
            return carry

        lax.fori_loop(0, nfull // 2, pair, 0)
        lwt(idx0, rows0, si0, sm0)
        pltpu.sync_copy(rows0, agg_sh.at[idx0], add=True)
        offt = pl.multiple_of(wbase + nfull * _CH, 8)
        pltpu.sync_copy(dst_hbm.at[pl.ds(offt, tail)], idx_t)
        pltpu.sync_copy(msg_hbm.at[pl.ds(offt, tail)], rows_t)
        pltpu.sync_copy(rows_t, agg_sh.at[idx_t], add=True)
        plsc.subcore_barrier()
        ro = pl.multiple_of(s * 632, 8)
        po = pl.multiple_of(c * _N + s * 632, 8)
        pltpu.sync_copy(agg_sh.at[pl.ds(ro, 520)],
                        parts_hbm.at[pl.ds(po, 520)])

        @pl.when(s < _NS - 1)
        def _out_rest():
            pltpu.sync_copy(
                agg_sh.at[pl.ds(pl.multiple_of(ro + 520, 8), 112)],
                parts_hbm.at[pl.ds(pl.multiple_of(po + 520, 8), 112)])

    return body, tail


@functools.lru_cache(maxsize=None)
def _sc_kernels():
    mesh = plsc.VectorSubcoreMesh(core_axis_name="c", subcore_axis_name="s")
    params = pltpu.CompilerParams(use_tc_tiling_on_sc=True)
    kernels = []
    for base, n_edges in _HALVES:
        epw = n_edges // _NW
        gbody, gtail = _make_gather_body(base, epw)
        gather = pl.kernel(
            gbody,
            mesh=mesh,
            compiler_params=params,
            out_type=jax.ShapeDtypeStruct((_E, _W), jnp.float32),
            scratch_types=[
                pltpu.VMEM((epw,), jnp.int32),
                pltpu.VMEM((_CH, _W), jnp.float32),
                pltpu.VMEM((_CH, _W), jnp.float32),
                pltpu.VMEM((max(gtail, 8), _W), jnp.float32),
                pltpu.SemaphoreType.DMA,
                pltpu.SemaphoreType.DMA,
            ],
        )
        sbody, stail = _make_scatter_body(base, epw)
        scatter = pl.kernel(
            sbody,
            mesh=mesh,
            compiler_params=params,
            out_type=jax.ShapeDtypeStruct((2 * _N, _W), jnp.float32),
            scratch_types=[
                pltpu.VMEM((_CH,), jnp.int32),
                pltpu.VMEM((_CH,), jnp.int32),
                pltpu.VMEM((_CH, _W), jnp.float32),
                pltpu.VMEM((_CH, _W), jnp.float32),
                pltpu.VMEM((max(stail, 8),), jnp.int32),
                pltpu.VMEM((max(stail, 8), _W), jnp.float32),
                pltpu.VMEM_SHARED((_N, _W), jnp.float32),
                pltpu.SemaphoreType.DMA,
                pltpu.SemaphoreType.DMA,
                pltpu.SemaphoreType.DMA,
                pltpu.SemaphoreType.DMA,
            ],
        )
        kernels.append((gather, scatter))
    return kernels


_BE = 8192
_BH = 256


def _edge_half(ea, xj, W2T):
    eaT = ea.T.astype(jnp.float32)
    xjT = xj.T
    rep = jnp.broadcast_to(xjT[:, None, :],
                           (_C, _C, _BH)).reshape(_C * _C, _BH)
    eaT_t = jnp.tile(eaT, (_C, 1))
    u = (rep * eaT_t).astype(jnp.bfloat16)
    u_full = jnp.concatenate([u, xjT.astype(jnp.bfloat16)], axis=0)
    msgT = jnp.dot(W2T, u_full,
                   preferred_element_type=jnp.float32)
    return msgT.T


def _edge_body(ea_ref, xj_ref, W2T_ref, msg_ref):
    W2T = W2T_ref[...]
    zer = jnp.zeros((_BH, _W - _C), jnp.float32)
    for h in range(_BE // _BH):
        sl = pl.ds(h * _BH, _BH)
        m = _edge_half(ea_ref[sl, :], xj_ref[sl, :][:, :_C], W2T)
        msg_ref[sl, :] = jnp.concatenate([m, zer], axis=1)


def _edge_call(ea, xj, W2T_bf16, blk0, n_edges):
    bmap = lambda i, b=blk0: (i + b, 0)
    return pl.pallas_call(
        _edge_body,
        grid=(pl.cdiv(n_edges, _BE),),
        in_specs=[
            pl.BlockSpec((_BE, _C), bmap),
            pl.BlockSpec((_BE, _W), bmap),
            pl.BlockSpec((_C, _C * _C + _C), lambda i: (0, 0)),
        ],
        out_specs=pl.BlockSpec((_BE, _W), bmap),
        out_shape=jax.ShapeDtypeStruct((_E, _W), jnp.float32),
    )(ea, xj, W2T_bf16)


_BN = 2000


def _combine_body(p_ref, x_ref, lwT_ref, b_ref, o_ref):
    s_ = (p_ref[0][:, :_C] + p_ref[1][:, :_C] + b_ref[...]
          + jnp.dot(x_ref[...][:, :_C], lwT_ref[...],
                    preferred_element_type=jnp.float32))
    h = jnp.maximum(s_, 0.0)
    o_ref[...] = jnp.concatenate(
        [h, jnp.zeros((_BN, _W - _C), jnp.float32)], axis=1)


def _combine_call(parts, x, lwT, b_row):
    return pl.pallas_call(
        _combine_body,
        grid=(_N // _BN,),
        in_specs=[
            pl.BlockSpec((2, _BN, _W), lambda i: (0, i, 0)),
            pl.BlockSpec((_BN, _W), lambda i: (i, 0)),
            pl.BlockSpec((_C, _C), lambda i: (0, 0)),
            pl.BlockSpec((1, _C), lambda i: (0, 0)),
        ],
        out_specs=pl.BlockSpec((_BN, _W), lambda i: (i, 0)),
        out_shape=jax.ShapeDtypeStruct((_N, _W), jnp.float32),
    )(parts, x, lwT, b_row)


def _combine_final_body(p_ref, x_ref, lwT_ref, b_ref, woT_ref, bo_ref, o_ref):
    s_ = (p_ref[0][:, :_C] + p_ref[1][:, :_C] + b_ref[...]
          + jnp.dot(x_ref[...][:, :_C], lwT_ref[...],
                    preferred_element_type=jnp.float32))
    h = jnp.maximum(s_, 0.0)
    o_ref[...] = jnp.dot(h, woT_ref[...],
                         preferred_element_type=jnp.float32) + bo_ref[...]


def _combine_final_call(parts, x, lwT, b_row, woT, bo_row):
    return pl.pallas_call(
        _combine_final_body,
        grid=(_N // _BN,),
        in_specs=[
            pl.BlockSpec((2, _BN, _W), lambda i: (0, i, 0)),
            pl.BlockSpec((_BN, _W), lambda i: (i, 0)),
            pl.BlockSpec((_C, _C), lambda i: (0, 0)),
            pl.BlockSpec((1, _C), lambda i: (0, 0)),
            pl.BlockSpec((_C, _C), lambda i: (0, 0)),
            pl.BlockSpec((1, _C), lambda i: (0, 0)),
        ],
        out_specs=pl.BlockSpec((_BN, _C), lambda i: (i, 0)),
        out_shape=jax.ShapeDtypeStruct((_N, _C), jnp.float32),
    )(parts, x, lwT, b_row, woT, bo_row)


def kernel(feature, edge_index, edge_attr, nn_W0, nn_b0, lin_W0, bias0,
           nn_W1, nn_b1, lin_W1, bias1, W_out, b_out):
    src = edge_index[0]
    dst = edge_index[1]
    zero_init = jnp.zeros((632, _W), jnp.float32)
    halves = _sc_kernels()

    def layer(x, nn_W, nn_b):
        W2T = nn_W.reshape(_C, _C, _C).transpose(1, 0, 2).reshape(_C, _C * _C)
        nbmT = nn_b.reshape(_C, _C).T
        W2T = jnp.concatenate([W2T, nbmT], axis=1).astype(jnp.bfloat16)
        gather, scatter = halves[0]
        xj = gather(x, src)
        msg = _edge_call(ea_b, xj, W2T, 0, _E)
        return scatter(msg, dst, zero_init).reshape(2, _N, _W)

    ea_b = edge_attr.astype(jnp.bfloat16)
    feat_pad = jnp.concatenate(
        [feature, jnp.zeros((_N, _W - _C), jnp.float32)], axis=1)
    p0 = layer(feat_pad, nn_W0, nn_b0)
    x1 = _combine_call(p0, feat_pad, lin_W0.T, bias0.reshape(1, _C))
    p1 = layer(x1, nn_W1, nn_b1)
    out = _combine_final_call(p1, x1, lin_W1.T, bias1.reshape(1, _C),
                              W_out.T, b_out.reshape(1, _C))
    return out
